# baseline (speedup 1.0000x reference)
import functools

import jax
import jax.numpy as jnp
from jax import lax
from jax.experimental import pallas as pl
from jax.experimental.pallas import tpu as pltpu
from jax.experimental.pallas import tpu_sc as plsc

_VOCAB = 1000000
_DIM = 64
_PAIR = 2 * _DIM
_BATCH = 16384
_LANES = 16

_NC = 2
_NS = 16
_NW = _NC * _NS
_BPW = _BATCH // _NW
_CHUNK = 128
_NCHUNK = _BPW // _CHUNK
_ROWS = _BATCH // 128
_NBUF = 2


def _lane_permute(x, idx):
    return lax.gather(
        x, idx[:, None],
        lax.GatherDimensionNumbers(
            offset_dims=(), collapsed_slice_dims=(0,), start_index_map=(0,)),
        slice_sizes=(1,),
        mode=lax.GatherScatterMode.PROMISE_IN_BOUNDS)


def _sc_body(t_hbm, c_hbm, tt_hbm, ct_hbm, out_hbm,
             tidx, cidx, tidx_g, cidx_g, trows, crows, scores, sem):
    wid = lax.axis_index("s") * _NC + lax.axis_index("c")
    base = wid * _NCHUNK

    pltpu.sync_copy(t_hbm.at[pl.ds(base, _NCHUNK)], tidx)
    pltpu.sync_copy(c_hbm.at[pl.ds(base, _NCHUNK)], cidx)

    for ci in range(_NCHUNK):
        for v in range(_CHUNK // _LANES):
            sl = pl.ds(v * _LANES, _LANES)
            tidx_g[ci, sl] = tidx[ci, sl] >> 1
            cidx_g[ci, sl] = cidx[ci, sl] >> 1

    def fire(ci, slot):
        return (pltpu.async_copy(tt_hbm.at[tidx_g.at[ci]], trows.at[slot],
                                 sem),
                pltpu.async_copy(ct_hbm.at[cidx_g.at[ci]], crows.at[slot],
                                 sem))

    lane = lax.iota(jnp.int32, _LANES)
    stages = [(lane ^ h, (lane & h) == 0) for h in (8, 4, 2, 1)]
    bitrev = (((lane & 1) << 3) | ((lane & 2) << 1)
              | ((lane & 4) >> 1) | ((lane & 8) >> 3))

    def merge(a, b, perm_h, mask_h):
        u = a + _lane_permute(a, perm_h)
        v = b + _lane_permute(b, perm_h)
        return jnp.where(mask_h, u, v)

    def tree(vecs):
        for perm_h, mask_h in stages:
            vecs = [merge(vecs[i], vecs[i + 1], perm_h, mask_h)
                    for i in range(0, len(vecs), 2)]
        return _lane_permute(vecs[0], bitrev)

    pending = fire(0, 0)
    for ci in range(_NCHUNK):
        slot = ci % _NBUF
        pending[0].wait()
        pending[1].wait()
        if ci + 1 < _NCHUNK:
            pending = fire(ci + 1, (ci + 1) % _NBUF)

        for g in range(_CHUNK // _LANES):
            sl = pl.ds(g * _LANES, _LANES)
            tvec = tidx[ci, sl]
            cvec = cidx[ci, sl]
            vecs = []
            for r in range(_LANES):
                j = g * _LANES + r
                toff = (tvec[r] & 1) * _DIM
                coff = (cvec[r] & 1) * _DIM
                acc = (trows[slot, j, pl.ds(toff, _LANES)]
                       * crows[slot, j, pl.ds(coff, _LANES)])
                for k in range(1, _DIM // _LANES):
                    acc = acc + (
                        trows[slot, j, pl.ds(toff + k * _LANES, _LANES)]
                        * crows[slot, j, pl.ds(coff + k * _LANES, _LANES)])
                vecs.append(acc)
            scores[ci, sl] = tree(vecs)

    pltpu.sync_copy(scores, out_hbm.at[pl.ds(base, _NCHUNK)])


@jax.jit
def _sc_scores(t_idx, c_idx, tt_pairs, ct_pairs):
    mesh = plsc.VectorSubcoreMesh(core_axis_name="c", subcore_axis_name="s")
    k = functools.partial(
        pl.kernel,
        mesh=mesh,
        out_type=jax.ShapeDtypeStruct((_ROWS, 128), jnp.float32),
        scratch_types=[
            pltpu.VMEM((_NCHUNK, _CHUNK), jnp.int32),
            pltpu.VMEM((_NCHUNK, _CHUNK), jnp.int32),
            pltpu.VMEM((_NCHUNK, _CHUNK), jnp.int32),
            pltpu.VMEM((_NCHUNK, _CHUNK), jnp.int32),
            pltpu.VMEM((_NBUF, _CHUNK, _PAIR), jnp.float32),
            pltpu.VMEM((_NBUF, _CHUNK, _PAIR), jnp.float32),
            pltpu.VMEM((_NCHUNK, _CHUNK), jnp.float32),
            pltpu.SemaphoreType.DMA,
        ],
    )(_sc_body)
    return k(t_idx, c_idx, tt_pairs, ct_pairs)


def kernel(target, context, target_table, context_table):
    t_idx = target.astype(jnp.int32).reshape(_ROWS, 128)
    c_idx = context.astype(jnp.int32).reshape(_ROWS, 128)
    tt_pairs = target_table.reshape(_VOCAB // 2, _PAIR)
    ct_pairs = context_table.reshape(_VOCAB // 2, _PAIR)
    out = _sc_scores(t_idx, c_idx, tt_pairs, ct_pairs)
    return out.reshape(_BATCH)
